# Initial kernel scaffold; baseline (speedup 1.0000x reference)
import jax
import jax.numpy as jnp
from jax.experimental import pallas as pl


def kernel(x_n1, x_n2, ei_n1_n2, ei_n2_n1, edge_index, params):
    raise NotImplementedError("write your pallas kernel here")


# speedup vs baseline: 12.3219x; 12.3219x over previous
import functools

import numpy as np
import jax
import jax.numpy as jnp
from jax import lax
from jax.experimental import pallas as pl
from jax.experimental.pallas import tpu as pltpu
from jax.experimental.pallas import tpu_sc as plsc

N = 25000
D_FEAT = 128
HID = 128
H = 4
DH = 32
E = 400000

NC = 2
NS = 16
NW = NC * NS

CH = 128
NCHUNK = E // CH

NBLK = 196
ECH = 256

_f32 = jnp.float32
_mesh = plsc.VectorSubcoreMesh(core_axis_name="c", subcore_axis_name="s")


def _mm_body(x_ref, w_ref, b_ref, o_ref, *, relu):
    y = jnp.dot(x_ref[...], w_ref[...], preferred_element_type=_f32) + b_ref[...]
    if relu:
        y = jnp.maximum(y, 0.0)
    o_ref[...] = y


def _mm(x, w, b, relu=False, bm=1000):
    m, k = x.shape
    n = w.shape[1]
    return pl.pallas_call(
        functools.partial(_mm_body, relu=relu),
        grid=(m // bm,),
        in_specs=[
            pl.BlockSpec((bm, k), lambda i: (i, 0)),
            pl.BlockSpec((k, n), lambda i: (0, 0)),
            pl.BlockSpec((1, n), lambda i: (0, 0)),
        ],
        out_specs=pl.BlockSpec((bm, n), lambda i: (i, 0)),
        out_shape=jax.ShapeDtypeStruct((m, n), _f32),
    )(x, w, b.reshape(1, n))


def _cat_body(x_ref, w_ref, b_ref, k_ref, q_ref, v_ref):
    y = jnp.dot(x_ref[...], w_ref[...], preferred_element_type=_f32) + b_ref[...]
    k_ref[...] = y[:, :HID]
    q_ref[...] = y[:, HID:2 * HID]
    v_ref[...] = y[:, 2 * HID:]


def _cat(x, w, b, bm=1000):
    m = x.shape[0]
    n = 3 * HID
    return pl.pallas_call(
        _cat_body,
        grid=(m // bm,),
        in_specs=[
            pl.BlockSpec((bm, HID), lambda i: (i, 0)),
            pl.BlockSpec((HID, n), lambda i: (0, 0)),
            pl.BlockSpec((1, n), lambda i: (0, 0)),
        ],
        out_specs=[
            pl.BlockSpec((bm, HID), lambda i: (i, 0)),
            pl.BlockSpec((bm, HID), lambda i: (i, 0)),
            pl.BlockSpec((bm, HID), lambda i: (i, 0)),
        ],
        out_shape=[jax.ShapeDtypeStruct((m, HID), _f32)] * 3,
    )(x, w, b.reshape(1, n))


def _exmsg_body(q_ref, k_ref, v_ref, ex_ref, msg_ref, *, bm):
    prod = q_ref[...] * k_ref[...]
    ex = jnp.concatenate(
        [jnp.sum(prod[:, DH * h:DH * (h + 1)], axis=1, keepdims=True)
         for h in range(H)], axis=1)
    ex = jnp.exp(jnp.minimum(ex, 75.0))
    ex_ref[...] = jnp.concatenate([ex, jnp.zeros((bm, 12), _f32)], axis=1)
    r = jnp.concatenate(
        [jnp.broadcast_to(ex[:, h:h + 1], (bm, DH)) for h in range(H)], axis=1)
    msg_ref[...] = v_ref[...] * r


def _exmsg(q_i, k_j, v_j, bm=1000):
    return pl.pallas_call(
        functools.partial(_exmsg_body, bm=bm),
        grid=(E // bm,),
        in_specs=[pl.BlockSpec((bm, HID), lambda i: (i, 0))] * 3,
        out_specs=[
            pl.BlockSpec((bm, 16), lambda i: (i, 0)),
            pl.BlockSpec((bm, HID), lambda i: (i, 0)),
        ],
        out_shape=[
            jax.ShapeDtypeStruct((E, 16), _f32),
            jax.ShapeDtypeStruct((E, HID), _f32),
        ],
    )(q_i, k_j, v_j)


def _seg_body(off_ref, msg_hbm, ex_hbm, dst_hbm, agg_ref, den_ref,
              msgb, exb, dstb, sem1, sem2, sem3):
    i = pl.program_id(0)
    lo = off_ref[i]
    hi = off_ref[i + 1]
    lo8 = (lo // 8) * 8
    nch = (hi - lo8 + ECH - 1) // ECH
    base = i * 128

    def body(j, carry):
        accm, accd = carry
        start0 = lo8 + j * ECH
        start = jnp.minimum(start0, E - ECH)
        cm = pltpu.make_async_copy(msg_hbm.at[pl.ds(start, ECH)], msgb, sem1)
        ce = pltpu.make_async_copy(ex_hbm.at[pl.ds(start, ECH)], exb, sem2)
        cd = pltpu.make_async_copy(dst_hbm.at[pl.ds(start, ECH)], dstb, sem3)
        cm.start()
        ce.start()
        cd.start()
        cm.wait()
        ce.wait()
        cd.wait()
        pos = start + lax.broadcasted_iota(jnp.int32, (ECH, 1), 0)
        keep = ((pos >= jnp.maximum(lo, start0))
                & (pos < jnp.minimum(hi, start0 + ECH)))
        oh = ((dstb[...] == base + lax.broadcasted_iota(jnp.int32, (1, 128), 1))
              & keep).astype(_f32)
        accm = accm + lax.dot_general(oh, msgb[...], (((0,), (0,)), ((), ())),
                                      preferred_element_type=_f32)
        accd = accd + lax.dot_general(oh, exb[...], (((0,), (0,)), ((), ())),
                                      preferred_element_type=_f32)
        return accm, accd

    accm, accd = lax.fori_loop(
        0, nch, body,
        (jnp.zeros((128, 128), _f32), jnp.zeros((128, 16), _f32)))
    agg_ref[...] = accm
    den_ref[...] = accd


def _seg_onehot(off, msg, ex, dst2d):
    grid_spec = pltpu.PrefetchScalarGridSpec(
        num_scalar_prefetch=1,
        grid=(NBLK,),
        in_specs=[
            pl.BlockSpec(memory_space=pltpu.HBM),
            pl.BlockSpec(memory_space=pltpu.HBM),
            pl.BlockSpec(memory_space=pltpu.HBM),
        ],
        out_specs=[
            pl.BlockSpec((128, 128), lambda i, off: (i, 0)),
            pl.BlockSpec((128, 16), lambda i, off: (i, 0)),
        ],
        scratch_shapes=[
            pltpu.VMEM((ECH, 128), _f32),
            pltpu.VMEM((ECH, 16), _f32),
            pltpu.VMEM((ECH, 1), jnp.int32),
            pltpu.SemaphoreType.DMA,
            pltpu.SemaphoreType.DMA,
            pltpu.SemaphoreType.DMA,
        ],
    )
    return pl.pallas_call(
        _seg_body,
        grid_spec=grid_spec,
        out_shape=[
            jax.ShapeDtypeStruct((NBLK * 128, 128), _f32),
            jax.ShapeDtypeStruct((NBLK * 128, 16), _f32),
        ],
    )(off, msg, ex, dst2d)


def _out_body(agg_ref, den_ref, x_ref, w_ref, b_ref, omb_ref, o_ref, *, bm):
    den = den_ref[...][:, :H]
    db = jnp.concatenate(
        [jnp.broadcast_to(den[:, h:h + 1], (bm, DH)) for h in range(H)], axis=1)
    a = jax.nn.gelu(agg_ref[...] / (db + 1e-16), approximate=True)
    o_ref[...] = (jnp.dot(a, w_ref[...], preferred_element_type=_f32)
                  + b_ref[...] + omb_ref[...] * x_ref[...])


def _out(agg, den, x, w_b, b_b, omb, bm=1000):
    return pl.pallas_call(
        functools.partial(_out_body, bm=bm),
        grid=(N // bm,),
        in_specs=[
            pl.BlockSpec((bm, HID), lambda i: (i, 0)),
            pl.BlockSpec((bm, 16), lambda i: (i, 0)),
            pl.BlockSpec((bm, HID), lambda i: (i, 0)),
            pl.BlockSpec((HID, HID), lambda i: (0, 0)),
            pl.BlockSpec((1, HID), lambda i: (0, 0)),
            pl.BlockSpec((1, HID), lambda i: (0, 0)),
        ],
        out_specs=pl.BlockSpec((bm, HID), lambda i: (i, 0)),
        out_shape=jax.ShapeDtypeStruct((N, HID), _f32),
    )(agg, den, x, w_b, b_b, omb)


def _ydot_body(a_ref, b_ref, o_ref):
    o_ref[...] = jnp.sum(a_ref[...] * b_ref[...], axis=1, keepdims=True)


def _ydot(a, b, bm=1000):
    d = a.shape[1]
    return pl.pallas_call(
        _ydot_body,
        grid=(E // bm,),
        in_specs=[pl.BlockSpec((bm, d), lambda i: (i, 0))] * 2,
        out_specs=pl.BlockSpec((bm, 1), lambda i: (i, 0)),
        out_shape=jax.ShapeDtypeStruct((E, 1), _f32),
    )(a, b)


@functools.partial(
    pl.kernel, mesh=_mesh,
    out_type=[jax.ShapeDtypeStruct((E, HID), _f32)] * 3,
    scratch_types=[
        pltpu.VMEM((CH,), jnp.int32),
        pltpu.VMEM((CH,), jnp.int32),
        pltpu.VMEM((CH, HID), _f32),
        pltpu.VMEM((CH, HID), _f32),
        pltpu.VMEM((CH, HID), _f32),
        pltpu.SemaphoreType.DMA,
    ],
)
def _gather3(ktab, qtab, vtab, src, dst, ko, qo, vo, sidx, didx, kb, qb, vb, sem):
    wid = lax.axis_index("s") * NC + lax.axis_index("c")

    @pl.loop(wid, NCHUNK, step=NW)
    def _chunk(c):
        base = c * CH
        pltpu.sync_copy(src.at[pl.ds(base, CH)], sidx)
        pltpu.sync_copy(dst.at[pl.ds(base, CH)], didx)
        d1 = pltpu.async_copy(ktab.at[sidx], kb, sem)
        d2 = pltpu.async_copy(vtab.at[sidx], vb, sem)
        d3 = pltpu.async_copy(qtab.at[didx], qb, sem)
        d1.wait()
        d2.wait()
        d3.wait()
        pltpu.sync_copy(kb, ko.at[pl.ds(base, CH)])
        pltpu.sync_copy(qb, qo.at[pl.ds(base, CH)])
        pltpu.sync_copy(vb, vo.at[pl.ds(base, CH)])


@functools.partial(
    pl.kernel, mesh=_mesh,
    out_type=[jax.ShapeDtypeStruct((E, 256), _f32)] * 2,
    scratch_types=[
        pltpu.VMEM((CH,), jnp.int32),
        pltpu.VMEM((CH,), jnp.int32),
        pltpu.VMEM((CH, 256), _f32),
        pltpu.VMEM((CH, 256), _f32),
        pltpu.SemaphoreType.DMA,
    ],
)
def _gather2(mtab, dtab, midx_h, didx_h, mo, do, midx, didx, mb, db, sem):
    wid = lax.axis_index("s") * NC + lax.axis_index("c")

    @pl.loop(wid, NCHUNK, step=NW)
    def _chunk(c):
        base = c * CH
        pltpu.sync_copy(midx_h.at[pl.ds(base, CH)], midx)
        pltpu.sync_copy(didx_h.at[pl.ds(base, CH)], didx)
        d1 = pltpu.async_copy(mtab.at[midx], mb, sem)
        d2 = pltpu.async_copy(dtab.at[didx], db, sem)
        d1.wait()
        d2.wait()
        pltpu.sync_copy(mb, mo.at[pl.ds(base, CH)])
        pltpu.sync_copy(db, do.at[pl.ds(base, CH)])


def _block_diag(mats):
    return jax.scipy.linalg.block_diag(*[mats[h] for h in range(H)])


def kernel(x_n1, x_n2, ei_n1_n2, ei_n2_n1, edge_index, params):
    p = params

    x = {
        "n1": _mm(x_n1, p["lin_n1_w"], p["lin_n1_b"], relu=True),
        "n2": _mm(x_n2, p["lin_n2_w"], p["lin_n2_b"], relu=True),
    }

    edge = {}
    for key, ei in ((("n1", "n2"), ei_n1_n2), (("n2", "n1"), ei_n2_n1)):
        src = ei[0].astype(jnp.int32)
        dst = ei[1].astype(jnp.int32)
        perm = jnp.argsort(dst)
        srcp = src[perm]
        dstp = dst[perm]
        off = jnp.searchsorted(dstp, jnp.arange(NBLK + 1, dtype=jnp.int32) * 128
                               ).astype(jnp.int32)
        edge[key] = (srcp, dstp, dstp.reshape(E, 1), off)

    et_of = {"n1": "n1_to_n2", "n2": "n2_to_n1"}
    layer_outs = []
    for l in range(2):
        krel, q, vrel = {}, {}, {}
        for t in ("n1", "n2"):
            et = et_of[t]
            w = p[f"l{l}_kqv_{t}_w"]
            b = p[f"l{l}_kqv_{t}_b"]
            wk, wq, wv = w[:, :HID], w[:, HID:2 * HID], w[:, 2 * HID:]
            bk, bq, bv = b[:HID], b[HID:2 * HID], b[2 * HID:]
            a_s = p[f"l{l}_arel_{et}"] * (
                p[f"l{l}_prel_{et}"][:, None, None] / np.sqrt(DH))
            bda = _block_diag(a_s)
            bdm = _block_diag(p[f"l{l}_mrel_{et}"])
            wcat = jnp.concatenate([wk @ bda, wq, wv @ bdm], axis=1)
            bcat = jnp.concatenate([bk @ bda, bq, bv @ bdm])
            krel[t], q[t], vrel[t] = _cat(x[t], wcat, bcat)

        agg, den = {}, {}
        for (s_t, d_t) in (("n1", "n2"), ("n2", "n1")):
            srcp, dstp, dst2d, off = edge[(s_t, d_t)]
            k_j, q_i, v_j = _gather3(krel[s_t], q[d_t], vrel[s_t], srcp, dstp)
            ex, msg = _exmsg(q_i, k_j, v_j)
            agg_p, den_p = _seg_onehot(off, msg, ex, dst2d)
            agg[d_t] = agg_p[:N]
            den[d_t] = den_p[:N]

        newx = {}
        for t in ("n1", "n2"):
            beta = jax.nn.sigmoid(p[f"l{l}_skip_{t}"])
            w2 = p[f"l{l}_out_{t}_w"] * beta
            b2 = (p[f"l{l}_out_{t}_b"] * beta).reshape(1, HID)
            omb = jnp.full((1, HID), 1.0, _f32) * (1.0 - beta)
            newx[t] = _out(agg[t], den[t], x[t], w2, b2, omb)
        x = newx
        layer_outs.append(dict(x))

    em = jnp.concatenate([layer_outs[0]["n1"], layer_outs[1]["n1"]], axis=1)
    ed = jnp.concatenate([layer_outs[0]["n2"], layer_outs[1]["n2"]], axis=1)
    m_idx = edge_index[0].astype(jnp.int32)
    d_idx = edge_index[1].astype(jnp.int32)
    em_e, ed_e = _gather2(em, ed, m_idx, d_idx)
    return _ydot(em_e, ed_e)


# speedup vs baseline: 19.4489x; 1.5784x over previous
import functools

import numpy as np
import jax
import jax.numpy as jnp
from jax import lax
from jax.experimental import pallas as pl
from jax.experimental.pallas import tpu as pltpu
from jax.experimental.pallas import tpu_sc as plsc

N = 25000
D_FEAT = 128
HID = 128
H = 4
DH = 32
E = 400000

NC = 2
NS = 16
NW = NC * NS

CH = 128
NCHUNK = E // CH

NBLK = 196
ECH = 1024

_f32 = jnp.float32
_mesh = plsc.VectorSubcoreMesh(core_axis_name="c", subcore_axis_name="s")


def _mm_body(x_ref, w_ref, b_ref, o_ref, *, relu):
    y = jnp.dot(x_ref[...], w_ref[...], preferred_element_type=_f32) + b_ref[...]
    if relu:
        y = jnp.maximum(y, 0.0)
    o_ref[...] = y


def _mm(x, w, b, relu=False, bm=1000):
    m, k = x.shape
    n = w.shape[1]
    return pl.pallas_call(
        functools.partial(_mm_body, relu=relu),
        grid=(m // bm,),
        in_specs=[
            pl.BlockSpec((bm, k), lambda i: (i, 0)),
            pl.BlockSpec((k, n), lambda i: (0, 0)),
            pl.BlockSpec((1, n), lambda i: (0, 0)),
        ],
        out_specs=pl.BlockSpec((bm, n), lambda i: (i, 0)),
        out_shape=jax.ShapeDtypeStruct((m, n), _f32),
    )(x, w, b.reshape(1, n))


def _cat_body(x_ref, w_ref, b_ref, k_ref, q_ref, v_ref):
    y = jnp.dot(x_ref[...], w_ref[...], preferred_element_type=_f32) + b_ref[...]
    k_ref[...] = y[:, :HID]
    q_ref[...] = y[:, HID:2 * HID]
    v_ref[...] = y[:, 2 * HID:]


def _cat(x, w, b, bm=1000):
    m = x.shape[0]
    n = 3 * HID
    return pl.pallas_call(
        _cat_body,
        grid=(m // bm,),
        in_specs=[
            pl.BlockSpec((bm, HID), lambda i: (i, 0)),
            pl.BlockSpec((HID, n), lambda i: (0, 0)),
            pl.BlockSpec((1, n), lambda i: (0, 0)),
        ],
        out_specs=[
            pl.BlockSpec((bm, HID), lambda i: (i, 0)),
            pl.BlockSpec((bm, HID), lambda i: (i, 0)),
            pl.BlockSpec((bm, HID), lambda i: (i, 0)),
        ],
        out_shape=[jax.ShapeDtypeStruct((m, HID), _f32)] * 3,
    )(x, w, b.reshape(1, n))


def _seg_body(off_ref, q_hbm, k_hbm, v_hbm, dst_hbm, agg_ref, den_ref,
              qb, kb, vb, db, sq, sk, sv, sd):
    i = pl.program_id(0)
    lo = off_ref[i]
    hi = off_ref[i + 1]
    lo8 = (lo // 8) * 8
    nch = (hi - lo8 + ECH - 1) // ECH
    base = i * 128

    def _start(j, par):
        st = jnp.minimum(lo8 + j * ECH, E - ECH)
        pltpu.make_async_copy(q_hbm.at[pl.ds(st, ECH)], qb.at[par], sq.at[par]).start()
        pltpu.make_async_copy(k_hbm.at[pl.ds(st, ECH)], kb.at[par], sk.at[par]).start()
        pltpu.make_async_copy(v_hbm.at[pl.ds(st, ECH)], vb.at[par], sv.at[par]).start()
        pltpu.make_async_copy(dst_hbm.at[pl.ds(st, ECH)], db.at[par], sd.at[par]).start()

    @pl.when(nch > 0)
    def _prologue():
        _start(0, 0)

    def body(j, carry):
        accm, accd = carry
        par = lax.rem(j, 2)

        @pl.when(j + 1 < nch)
        def _next():
            _start(j + 1, lax.rem(j + 1, 2))

        pltpu.make_async_copy(q_hbm.at[pl.ds(0, ECH)], qb.at[par], sq.at[par]).wait()
        pltpu.make_async_copy(k_hbm.at[pl.ds(0, ECH)], kb.at[par], sk.at[par]).wait()
        pltpu.make_async_copy(v_hbm.at[pl.ds(0, ECH)], vb.at[par], sv.at[par]).wait()
        pltpu.make_async_copy(dst_hbm.at[pl.ds(0, ECH)], db.at[par], sd.at[par]).wait()

        start0 = lo8 + j * ECH
        start = jnp.minimum(start0, E - ECH)
        prod = qb[par] * kb[par]
        ex = jnp.concatenate(
            [jnp.sum(prod[:, DH * h:DH * (h + 1)], axis=1, keepdims=True)
             for h in range(H)], axis=1)
        ex = jnp.exp(jnp.minimum(ex, 75.0))
        r = jnp.concatenate(
            [jnp.broadcast_to(ex[:, h:h + 1], (ECH, DH)) for h in range(H)],
            axis=1)
        msg = vb[par] * r
        exw = jnp.concatenate([ex, jnp.zeros((ECH, 12), _f32)], axis=1)

        pos = start + lax.broadcasted_iota(jnp.int32, (ECH, 1), 0)
        keep = ((pos >= jnp.maximum(lo, start0))
                & (pos < jnp.minimum(hi, start0 + ECH)))
        oh = ((db[par] == base + lax.broadcasted_iota(jnp.int32, (1, 128), 1))
              & keep).astype(_f32)
        accm = accm + lax.dot_general(oh, msg, (((0,), (0,)), ((), ())),
                                      preferred_element_type=_f32)
        accd = accd + lax.dot_general(oh, exw, (((0,), (0,)), ((), ())),
                                      preferred_element_type=_f32)
        return accm, accd

    accm, accd = lax.fori_loop(
        0, nch, body,
        (jnp.zeros((128, 128), _f32), jnp.zeros((128, 16), _f32)))
    agg_ref[...] = accm
    den_ref[...] = accd


def _seg_onehot(off, q_i, k_j, v_j, dst2d):
    grid_spec = pltpu.PrefetchScalarGridSpec(
        num_scalar_prefetch=1,
        grid=(NBLK,),
        in_specs=[
            pl.BlockSpec(memory_space=pltpu.HBM),
            pl.BlockSpec(memory_space=pltpu.HBM),
            pl.BlockSpec(memory_space=pltpu.HBM),
            pl.BlockSpec(memory_space=pltpu.HBM),
        ],
        out_specs=[
            pl.BlockSpec((128, 128), lambda i, off: (i, 0)),
            pl.BlockSpec((128, 16), lambda i, off: (i, 0)),
        ],
        scratch_shapes=[
            pltpu.VMEM((2, ECH, HID), _f32),
            pltpu.VMEM((2, ECH, HID), _f32),
            pltpu.VMEM((2, ECH, HID), _f32),
            pltpu.VMEM((2, ECH, 1), jnp.int32),
            pltpu.SemaphoreType.DMA((2,)),
            pltpu.SemaphoreType.DMA((2,)),
            pltpu.SemaphoreType.DMA((2,)),
            pltpu.SemaphoreType.DMA((2,)),
        ],
    )
    return pl.pallas_call(
        _seg_body,
        grid_spec=grid_spec,
        out_shape=[
            jax.ShapeDtypeStruct((NBLK * 128, 128), _f32),
            jax.ShapeDtypeStruct((NBLK * 128, 16), _f32),
        ],
    )(off, q_i, k_j, v_j, dst2d)


def _out_body(agg_ref, den_ref, x_ref, w_ref, b_ref, omb_ref, o_ref, *, bm):
    den = den_ref[...][:, :H]
    db = jnp.concatenate(
        [jnp.broadcast_to(den[:, h:h + 1], (bm, DH)) for h in range(H)], axis=1)
    a = jax.nn.gelu(agg_ref[...] / (db + 1e-16), approximate=True)
    o_ref[...] = (jnp.dot(a, w_ref[...], preferred_element_type=_f32)
                  + b_ref[...] + omb_ref[...] * x_ref[...])


def _out(agg, den, x, w_b, b_b, omb, bm=1000):
    return pl.pallas_call(
        functools.partial(_out_body, bm=bm),
        grid=(N // bm,),
        in_specs=[
            pl.BlockSpec((bm, HID), lambda i: (i, 0)),
            pl.BlockSpec((bm, 16), lambda i: (i, 0)),
            pl.BlockSpec((bm, HID), lambda i: (i, 0)),
            pl.BlockSpec((HID, HID), lambda i: (0, 0)),
            pl.BlockSpec((1, HID), lambda i: (0, 0)),
            pl.BlockSpec((1, HID), lambda i: (0, 0)),
        ],
        out_specs=pl.BlockSpec((bm, HID), lambda i: (i, 0)),
        out_shape=jax.ShapeDtypeStruct((N, HID), _f32),
    )(agg, den, x, w_b, b_b, omb)


def _ydot_body(a_ref, b_ref, o_ref):
    o_ref[...] = jnp.sum(a_ref[...] * b_ref[...], axis=1, keepdims=True)


def _ydot(a, b, bm=1000):
    d = a.shape[1]
    return pl.pallas_call(
        _ydot_body,
        grid=(E // bm,),
        in_specs=[pl.BlockSpec((bm, d), lambda i: (i, 0))] * 2,
        out_specs=pl.BlockSpec((bm, 1), lambda i: (i, 0)),
        out_shape=jax.ShapeDtypeStruct((E, 1), _f32),
    )(a, b)


@functools.partial(
    pl.kernel, mesh=_mesh,
    out_type=[jax.ShapeDtypeStruct((E, HID), _f32)] * 3,
    scratch_types=[
        pltpu.VMEM((CH,), jnp.int32),
        pltpu.VMEM((CH,), jnp.int32),
        pltpu.VMEM((CH, HID), _f32),
        pltpu.VMEM((CH, HID), _f32),
        pltpu.VMEM((CH, HID), _f32),
        pltpu.SemaphoreType.DMA,
    ],
)
def _gather3(ktab, qtab, vtab, src, dst, ko, qo, vo, sidx, didx, kb, qb, vb, sem):
    wid = lax.axis_index("s") * NC + lax.axis_index("c")

    @pl.loop(wid, NCHUNK, step=NW)
    def _chunk(c):
        base = c * CH
        pltpu.sync_copy(src.at[pl.ds(base, CH)], sidx)
        pltpu.sync_copy(dst.at[pl.ds(base, CH)], didx)
        d1 = pltpu.async_copy(ktab.at[sidx], kb, sem)
        d2 = pltpu.async_copy(vtab.at[sidx], vb, sem)
        d3 = pltpu.async_copy(qtab.at[didx], qb, sem)
        d1.wait()
        d2.wait()
        d3.wait()
        pltpu.sync_copy(kb, ko.at[pl.ds(base, CH)])
        pltpu.sync_copy(qb, qo.at[pl.ds(base, CH)])
        pltpu.sync_copy(vb, vo.at[pl.ds(base, CH)])


@functools.partial(
    pl.kernel, mesh=_mesh,
    out_type=[jax.ShapeDtypeStruct((E, 256), _f32)] * 2,
    scratch_types=[
        pltpu.VMEM((CH,), jnp.int32),
        pltpu.VMEM((CH,), jnp.int32),
        pltpu.VMEM((CH, 256), _f32),
        pltpu.VMEM((CH, 256), _f32),
        pltpu.SemaphoreType.DMA,
    ],
)
def _gather2(mtab, dtab, midx_h, didx_h, mo, do, midx, didx, mb, db, sem):
    wid = lax.axis_index("s") * NC + lax.axis_index("c")

    @pl.loop(wid, NCHUNK, step=NW)
    def _chunk(c):
        base = c * CH
        pltpu.sync_copy(midx_h.at[pl.ds(base, CH)], midx)
        pltpu.sync_copy(didx_h.at[pl.ds(base, CH)], didx)
        d1 = pltpu.async_copy(mtab.at[midx], mb, sem)
        d2 = pltpu.async_copy(dtab.at[didx], db, sem)
        d1.wait()
        d2.wait()
        pltpu.sync_copy(mb, mo.at[pl.ds(base, CH)])
        pltpu.sync_copy(db, do.at[pl.ds(base, CH)])


def _block_diag(mats):
    return jax.scipy.linalg.block_diag(*[mats[h] for h in range(H)])


def kernel(x_n1, x_n2, ei_n1_n2, ei_n2_n1, edge_index, params):
    p = params

    x = {
        "n1": _mm(x_n1, p["lin_n1_w"], p["lin_n1_b"], relu=True),
        "n2": _mm(x_n2, p["lin_n2_w"], p["lin_n2_b"], relu=True),
    }

    edge = {}
    for key, ei in ((("n1", "n2"), ei_n1_n2), (("n2", "n1"), ei_n2_n1)):
        src = ei[0].astype(jnp.int32)
        dst = ei[1].astype(jnp.int32)
        perm = jnp.argsort(dst)
        srcp = src[perm]
        dstp = dst[perm]
        off = jnp.searchsorted(dstp, jnp.arange(NBLK + 1, dtype=jnp.int32) * 128
                               ).astype(jnp.int32)
        edge[key] = (srcp, dstp, dstp.reshape(E, 1), off)

    et_of = {"n1": "n1_to_n2", "n2": "n2_to_n1"}
    layer_outs = []
    for l in range(2):
        krel, q, vrel = {}, {}, {}
        for t in ("n1", "n2"):
            et = et_of[t]
            w = p[f"l{l}_kqv_{t}_w"]
            b = p[f"l{l}_kqv_{t}_b"]
            wk, wq, wv = w[:, :HID], w[:, HID:2 * HID], w[:, 2 * HID:]
            bk, bq, bv = b[:HID], b[HID:2 * HID], b[2 * HID:]
            a_s = p[f"l{l}_arel_{et}"] * (
                p[f"l{l}_prel_{et}"][:, None, None] / np.sqrt(DH))
            bda = _block_diag(a_s)
            bdm = _block_diag(p[f"l{l}_mrel_{et}"])
            wcat = jnp.concatenate([wk @ bda, wq, wv @ bdm], axis=1)
            bcat = jnp.concatenate([bk @ bda, bq, bv @ bdm])
            krel[t], q[t], vrel[t] = _cat(x[t], wcat, bcat)

        agg, den = {}, {}
        for (s_t, d_t) in (("n1", "n2"), ("n2", "n1")):
            srcp, dstp, dst2d, off = edge[(s_t, d_t)]
            k_j, q_i, v_j = _gather3(krel[s_t], q[d_t], vrel[s_t], srcp, dstp)
            agg_p, den_p = _seg_onehot(off, q_i, k_j, v_j, dst2d)
            agg[d_t] = agg_p[:N]
            den[d_t] = den_p[:N]

        newx = {}
        for t in ("n1", "n2"):
            beta = jax.nn.sigmoid(p[f"l{l}_skip_{t}"])
            w2 = p[f"l{l}_out_{t}_w"] * beta
            b2 = (p[f"l{l}_out_{t}_b"] * beta).reshape(1, HID)
            omb = jnp.full((1, HID), 1.0, _f32) * (1.0 - beta)
            newx[t] = _out(agg[t], den[t], x[t], w2, b2, omb)
        x = newx
        layer_outs.append(dict(x))

    em = jnp.concatenate([layer_outs[0]["n1"], layer_outs[1]["n1"]], axis=1)
    ed = jnp.concatenate([layer_outs[0]["n2"], layer_outs[1]["n2"]], axis=1)
    m_idx = edge_index[0].astype(jnp.int32)
    d_idx = edge_index[1].astype(jnp.int32)
    em_e, ed_e = _gather2(em, ed, m_idx, d_idx)
    return _ydot(em_e, ed_e)


# speedup vs baseline: 19.9010x; 1.0232x over previous
import functools

import numpy as np
import jax
import jax.numpy as jnp
from jax import lax
from jax.experimental import pallas as pl
from jax.experimental.pallas import tpu as pltpu
from jax.experimental.pallas import tpu_sc as plsc

N = 25000
D_FEAT = 128
HID = 128
H = 4
DH = 32
E = 400000

NC = 2
NS = 16
NW = NC * NS

CH = 128
NCHUNK = E // CH

NBLK = 196
ECH = 1024

_f32 = jnp.float32
_mesh = plsc.VectorSubcoreMesh(core_axis_name="c", subcore_axis_name="s")


def _mm_body(x_ref, w_ref, b_ref, o_ref, *, relu):
    y = jnp.dot(x_ref[...], w_ref[...], preferred_element_type=_f32) + b_ref[...]
    if relu:
        y = jnp.maximum(y, 0.0)
    o_ref[...] = y


def _mm(x, w, b, relu=False, bm=1000):
    m, k = x.shape
    n = w.shape[1]
    return pl.pallas_call(
        functools.partial(_mm_body, relu=relu),
        grid=(m // bm,),
        in_specs=[
            pl.BlockSpec((bm, k), lambda i: (i, 0)),
            pl.BlockSpec((k, n), lambda i: (0, 0)),
            pl.BlockSpec((1, n), lambda i: (0, 0)),
        ],
        out_specs=pl.BlockSpec((bm, n), lambda i: (i, 0)),
        out_shape=jax.ShapeDtypeStruct((m, n), _f32),
    )(x, w, b.reshape(1, n))


def _cat_body(x_ref, w_ref, b_ref, k_ref, q_ref, v_ref):
    y = jnp.dot(x_ref[...], w_ref[...], preferred_element_type=_f32) + b_ref[...]
    k_ref[...] = y[:, :HID]
    q_ref[...] = y[:, HID:2 * HID]
    v_ref[...] = y[:, 2 * HID:]


def _cat(x, w, b, bm=1000):
    m = x.shape[0]
    n = 3 * HID
    return pl.pallas_call(
        _cat_body,
        grid=(m // bm,),
        in_specs=[
            pl.BlockSpec((bm, HID), lambda i: (i, 0)),
            pl.BlockSpec((HID, n), lambda i: (0, 0)),
            pl.BlockSpec((1, n), lambda i: (0, 0)),
        ],
        out_specs=[
            pl.BlockSpec((bm, HID), lambda i: (i, 0)),
            pl.BlockSpec((bm, HID), lambda i: (i, 0)),
            pl.BlockSpec((bm, HID), lambda i: (i, 0)),
        ],
        out_shape=[jax.ShapeDtypeStruct((m, HID), _f32)] * 3,
    )(x, w, b.reshape(1, n))


def _seg_body(off_ref, q_hbm, k_hbm, v_hbm, dst_hbm, agg_ref, den_ref,
              qb, kb, vb, db, sq, sk, sv, sd):
    i = pl.program_id(0)
    lo = off_ref[i]
    hi = off_ref[i + 1]
    lo8 = (lo // 8) * 8
    nch = (hi - lo8 + ECH - 1) // ECH
    base = i * 128

    def _start(j, par):
        st = jnp.minimum(lo8 + j * ECH, E - ECH)
        pltpu.make_async_copy(q_hbm.at[pl.ds(st, ECH)], qb.at[par], sq.at[par]).start()
        pltpu.make_async_copy(k_hbm.at[pl.ds(st, ECH)], kb.at[par], sk.at[par]).start()
        pltpu.make_async_copy(v_hbm.at[pl.ds(st, ECH)], vb.at[par], sv.at[par]).start()
        pltpu.make_async_copy(dst_hbm.at[pl.ds(st, ECH)], db.at[par], sd.at[par]).start()

    @pl.when(nch > 0)
    def _prologue():
        _start(0, 0)

    def body(j, carry):
        accm, accd = carry
        par = lax.rem(j, 2)

        @pl.when(j + 1 < nch)
        def _next():
            _start(j + 1, lax.rem(j + 1, 2))

        pltpu.make_async_copy(q_hbm.at[pl.ds(0, ECH)], qb.at[par], sq.at[par]).wait()
        pltpu.make_async_copy(k_hbm.at[pl.ds(0, ECH)], kb.at[par], sk.at[par]).wait()
        pltpu.make_async_copy(v_hbm.at[pl.ds(0, ECH)], vb.at[par], sv.at[par]).wait()
        pltpu.make_async_copy(dst_hbm.at[pl.ds(0, ECH)], db.at[par], sd.at[par]).wait()

        start0 = lo8 + j * ECH
        start = jnp.minimum(start0, E - ECH)
        prod = qb[par] * kb[par]
        ex = jnp.concatenate(
            [jnp.sum(prod[:, DH * h:DH * (h + 1)], axis=1, keepdims=True)
             for h in range(H)], axis=1)
        ex = jnp.exp(jnp.minimum(ex, 75.0))
        r = jnp.concatenate(
            [jnp.broadcast_to(ex[:, h:h + 1], (ECH, DH)) for h in range(H)],
            axis=1)
        msg = vb[par] * r
        exw = jnp.concatenate([ex, jnp.zeros((ECH, 12), _f32)], axis=1)

        pos = start + lax.broadcasted_iota(jnp.int32, (ECH, 1), 0)
        keep = ((pos >= jnp.maximum(lo, start0))
                & (pos < jnp.minimum(hi, start0 + ECH)))
        oh = ((db[par] == base + lax.broadcasted_iota(jnp.int32, (1, 128), 1))
              & keep).astype(_f32)
        accm = accm + lax.dot_general(oh, msg, (((0,), (0,)), ((), ())),
                                      preferred_element_type=_f32)
        accd = accd + lax.dot_general(oh, exw, (((0,), (0,)), ((), ())),
                                      preferred_element_type=_f32)
        return accm, accd

    accm, accd = lax.fori_loop(
        0, nch, body,
        (jnp.zeros((128, 128), _f32), jnp.zeros((128, 16), _f32)))
    agg_ref[...] = accm
    den_ref[...] = accd


def _seg_onehot(off, q_i, k_j, v_j, dst2d):
    grid_spec = pltpu.PrefetchScalarGridSpec(
        num_scalar_prefetch=1,
        grid=(NBLK,),
        in_specs=[
            pl.BlockSpec(memory_space=pltpu.HBM),
            pl.BlockSpec(memory_space=pltpu.HBM),
            pl.BlockSpec(memory_space=pltpu.HBM),
            pl.BlockSpec(memory_space=pltpu.HBM),
        ],
        out_specs=[
            pl.BlockSpec((128, 128), lambda i, off: (i, 0)),
            pl.BlockSpec((128, 16), lambda i, off: (i, 0)),
        ],
        scratch_shapes=[
            pltpu.VMEM((2, ECH, HID), _f32),
            pltpu.VMEM((2, ECH, HID), _f32),
            pltpu.VMEM((2, ECH, HID), _f32),
            pltpu.VMEM((2, ECH, 1), jnp.int32),
            pltpu.SemaphoreType.DMA((2,)),
            pltpu.SemaphoreType.DMA((2,)),
            pltpu.SemaphoreType.DMA((2,)),
            pltpu.SemaphoreType.DMA((2,)),
        ],
    )
    return pl.pallas_call(
        _seg_body,
        grid_spec=grid_spec,
        out_shape=[
            jax.ShapeDtypeStruct((NBLK * 128, 128), _f32),
            jax.ShapeDtypeStruct((NBLK * 128, 16), _f32),
        ],
    )(off, q_i, k_j, v_j, dst2d)


def _out_body(agg_ref, den_ref, x_ref, w_ref, b_ref, omb_ref, o_ref, *, bm):
    den = den_ref[...][:, :H]
    db = jnp.concatenate(
        [jnp.broadcast_to(den[:, h:h + 1], (bm, DH)) for h in range(H)], axis=1)
    a = jax.nn.gelu(agg_ref[...] / (db + 1e-16), approximate=True)
    o_ref[...] = (jnp.dot(a, w_ref[...], preferred_element_type=_f32)
                  + b_ref[...] + omb_ref[...] * x_ref[...])


def _out(agg, den, x, w_b, b_b, omb, bm=1000):
    return pl.pallas_call(
        functools.partial(_out_body, bm=bm),
        grid=(N // bm,),
        in_specs=[
            pl.BlockSpec((bm, HID), lambda i: (i, 0)),
            pl.BlockSpec((bm, 16), lambda i: (i, 0)),
            pl.BlockSpec((bm, HID), lambda i: (i, 0)),
            pl.BlockSpec((HID, HID), lambda i: (0, 0)),
            pl.BlockSpec((1, HID), lambda i: (0, 0)),
            pl.BlockSpec((1, HID), lambda i: (0, 0)),
        ],
        out_specs=pl.BlockSpec((bm, HID), lambda i: (i, 0)),
        out_shape=jax.ShapeDtypeStruct((N, HID), _f32),
    )(agg, den, x, w_b, b_b, omb)


def _ydot_body(a_ref, b_ref, o_ref):
    o_ref[...] = jnp.sum(a_ref[...] * b_ref[...], axis=1, keepdims=True)


def _ydot(a, b, bm=1000):
    d = a.shape[1]
    return pl.pallas_call(
        _ydot_body,
        grid=(E // bm,),
        in_specs=[pl.BlockSpec((bm, d), lambda i: (i, 0))] * 2,
        out_specs=pl.BlockSpec((bm, 1), lambda i: (i, 0)),
        out_shape=jax.ShapeDtypeStruct((E, 1), _f32),
    )(a, b)


@functools.partial(
    pl.kernel, mesh=_mesh,
    out_type=[jax.ShapeDtypeStruct((E, HID), _f32)] * 3,
    scratch_types=[
        pltpu.VMEM((2, CH), jnp.int32),
        pltpu.VMEM((2, CH), jnp.int32),
        pltpu.VMEM((2, CH, HID), _f32),
        pltpu.VMEM((2, CH, HID), _f32),
        pltpu.VMEM((2, CH, HID), _f32),
        pltpu.SemaphoreType.DMA((2,)),
        pltpu.SemaphoreType.DMA((2,)),
    ],
)
def _gather3(ktab, qtab, vtab, src, dst, ko, qo, vo, sidx, didx, kb, qb, vb,
             gsem, wsem):
    wid = lax.axis_index("s") * NC + lax.axis_index("c")
    n = (NCHUNK - wid + NW - 1) // NW

    def _front(j, par):
        c = wid + j * NW
        base = c * CH
        pltpu.sync_copy(src.at[pl.ds(base, CH)], sidx.at[par])
        pltpu.sync_copy(dst.at[pl.ds(base, CH)], didx.at[par])
        pltpu.async_copy(ktab.at[sidx.at[par]], kb.at[par], gsem.at[par])
        pltpu.async_copy(vtab.at[sidx.at[par]], vb.at[par], gsem.at[par])
        pltpu.async_copy(qtab.at[didx.at[par]], qb.at[par], gsem.at[par])

    def _wait(sem, par):
        for _ in range(3):
            pltpu.make_async_copy(ko.at[pl.ds(0, CH)], kb.at[par], sem.at[par]).wait()

    _front(0, 0)

    @pl.loop(0, n)
    def _iter(j):
        par = lax.rem(j, 2)
        nxt = 1 - par
        _wait(gsem, par)
        c = wid + j * NW
        base = c * CH
        pltpu.async_copy(kb.at[par], ko.at[pl.ds(base, CH)], wsem.at[par])
        pltpu.async_copy(qb.at[par], qo.at[pl.ds(base, CH)], wsem.at[par])
        pltpu.async_copy(vb.at[par], vo.at[pl.ds(base, CH)], wsem.at[par])

        @pl.when(j + 1 < n)
        def _next():
            @pl.when(j >= 1)
            def _w():
                _wait(wsem, nxt)
            _front(j + 1, nxt)

    _wait(wsem, lax.rem(n - 1, 2))
    _wait(wsem, lax.rem(n, 2))


CH2 = 64
NCHUNK2 = E // CH2


@functools.partial(
    pl.kernel, mesh=_mesh,
    out_type=[jax.ShapeDtypeStruct((E, 256), _f32)] * 2,
    scratch_types=[
        pltpu.VMEM((2, CH2), jnp.int32),
        pltpu.VMEM((2, CH2), jnp.int32),
        pltpu.VMEM((2, CH2, 256), _f32),
        pltpu.VMEM((2, CH2, 256), _f32),
        pltpu.SemaphoreType.DMA((2,)),
        pltpu.SemaphoreType.DMA((2,)),
    ],
)
def _gather2(mtab, dtab, midx_h, didx_h, mo, do, midx, didx, mb, db, gsem, wsem):
    wid = lax.axis_index("s") * NC + lax.axis_index("c")
    n = (NCHUNK2 - wid + NW - 1) // NW

    def _front(j, par):
        base = (wid + j * NW) * CH2
        pltpu.sync_copy(midx_h.at[pl.ds(base, CH2)], midx.at[par])
        pltpu.sync_copy(didx_h.at[pl.ds(base, CH2)], didx.at[par])
        pltpu.async_copy(mtab.at[midx.at[par]], mb.at[par], gsem.at[par])
        pltpu.async_copy(dtab.at[didx.at[par]], db.at[par], gsem.at[par])

    def _wait(sem, par):
        for _ in range(2):
            pltpu.make_async_copy(mo.at[pl.ds(0, CH2)], mb.at[par], sem.at[par]).wait()

    _front(0, 0)

    @pl.loop(0, n)
    def _iter(j):
        par = lax.rem(j, 2)
        nxt = 1 - par
        _wait(gsem, par)
        base = (wid + j * NW) * CH2
        pltpu.async_copy(mb.at[par], mo.at[pl.ds(base, CH2)], wsem.at[par])
        pltpu.async_copy(db.at[par], do.at[pl.ds(base, CH2)], wsem.at[par])

        @pl.when(j + 1 < n)
        def _next():
            @pl.when(j >= 1)
            def _w():
                _wait(wsem, nxt)
            _front(j + 1, nxt)

    _wait(wsem, lax.rem(n - 1, 2))
    _wait(wsem, lax.rem(n, 2))


def _block_diag(mats):
    return jax.scipy.linalg.block_diag(*[mats[h] for h in range(H)])


def kernel(x_n1, x_n2, ei_n1_n2, ei_n2_n1, edge_index, params):
    p = params

    x = {
        "n1": _mm(x_n1, p["lin_n1_w"], p["lin_n1_b"], relu=True),
        "n2": _mm(x_n2, p["lin_n2_w"], p["lin_n2_b"], relu=True),
    }

    edge = {}
    for key, ei in ((("n1", "n2"), ei_n1_n2), (("n2", "n1"), ei_n2_n1)):
        src = ei[0].astype(jnp.int32)
        dst = ei[1].astype(jnp.int32)
        perm = jnp.argsort(dst)
        srcp = src[perm]
        dstp = dst[perm]
        off = jnp.searchsorted(dstp, jnp.arange(NBLK + 1, dtype=jnp.int32) * 128
                               ).astype(jnp.int32)
        edge[key] = (srcp, dstp, dstp.reshape(E, 1), off)

    et_of = {"n1": "n1_to_n2", "n2": "n2_to_n1"}
    layer_outs = []
    for l in range(2):
        krel, q, vrel = {}, {}, {}
        for t in ("n1", "n2"):
            et = et_of[t]
            w = p[f"l{l}_kqv_{t}_w"]
            b = p[f"l{l}_kqv_{t}_b"]
            wk, wq, wv = w[:, :HID], w[:, HID:2 * HID], w[:, 2 * HID:]
            bk, bq, bv = b[:HID], b[HID:2 * HID], b[2 * HID:]
            a_s = p[f"l{l}_arel_{et}"] * (
                p[f"l{l}_prel_{et}"][:, None, None] / np.sqrt(DH))
            bda = _block_diag(a_s)
            bdm = _block_diag(p[f"l{l}_mrel_{et}"])
            wcat = jnp.concatenate([wk @ bda, wq, wv @ bdm], axis=1)
            bcat = jnp.concatenate([bk @ bda, bq, bv @ bdm])
            krel[t], q[t], vrel[t] = _cat(x[t], wcat, bcat)

        agg, den = {}, {}
        for (s_t, d_t) in (("n1", "n2"), ("n2", "n1")):
            srcp, dstp, dst2d, off = edge[(s_t, d_t)]
            k_j, q_i, v_j = _gather3(krel[s_t], q[d_t], vrel[s_t], srcp, dstp)
            agg_p, den_p = _seg_onehot(off, q_i, k_j, v_j, dst2d)
            agg[d_t] = agg_p[:N]
            den[d_t] = den_p[:N]

        newx = {}
        for t in ("n1", "n2"):
            beta = jax.nn.sigmoid(p[f"l{l}_skip_{t}"])
            w2 = p[f"l{l}_out_{t}_w"] * beta
            b2 = (p[f"l{l}_out_{t}_b"] * beta).reshape(1, HID)
            omb = jnp.full((1, HID), 1.0, _f32) * (1.0 - beta)
            newx[t] = _out(agg[t], den[t], x[t], w2, b2, omb)
        x = newx
        layer_outs.append(dict(x))

    em = jnp.concatenate([layer_outs[0]["n1"], layer_outs[1]["n1"]], axis=1)
    ed = jnp.concatenate([layer_outs[0]["n2"], layer_outs[1]["n2"]], axis=1)
    m_idx = edge_index[0].astype(jnp.int32)
    d_idx = edge_index[1].astype(jnp.int32)
    em_e, ed_e = _gather2(em, ed, m_idx, d_idx)
    return _ydot(em_e, ed_e)
